# baseline (speedup 1.0000x reference)
import functools

import jax
import jax.numpy as jnp
from jax import lax
from jax.experimental import pallas as pl
from jax.experimental.pallas import tpu as pltpu
from jax.experimental.pallas import tpu_sc as plsc

N = 10000
E = 160000
D = 256
EPS = 1e-5

NC = 2
NS = 16
L = 16

CCH = 32
NCCH = D // CCH
C = 128
KB = 4
NCH = 80
EP = NS * NCH * C
TROWS = 30720
RPT = TROWS // NS
WB = 128


def _sc_edge_pass(h4, src_r, gidx_r, norm_r):
    mesh = plsc.VectorSubcoreMesh(
        core_axis_name="c", subcore_axis_name="s",
        num_cores=NC, num_subcores=NS)

    @functools.partial(
        pl.kernel,
        out_type=jax.ShapeDtypeStruct((TROWS, D), jnp.float32),
        mesh=mesh,
        scratch_types=[
            pltpu.VMEM((NCH, C), jnp.int32),
            pltpu.VMEM((NCH, C), jnp.int32),
            pltpu.VMEM((NCH * C,), jnp.float32),
            pltpu.VMEM((2 * KB, C, CCH), jnp.float32),
            pltpu.VMEM((WB, CCH), jnp.float32),
            pltpu.VMEM_SHARED((TROWS, CCH), jnp.float32),
            pltpu.SemaphoreType.DMA,
            pltpu.SemaphoreType.DMA,
        ],
        compiler_params=pltpu.CompilerParams(use_tc_tiling_on_sc=False),
    )
    def k(h4_hbm, src_hbm, gidx_hbm, norm_hbm, out_hbm,
          srcc, gidx, normv, rows, wb, acc, sem, sem2):
        core = lax.axis_index("c")
        tid = lax.axis_index("s")

        pltpu.sync_copy(gidx_hbm.at[tid], gidx)
        pltpu.sync_copy(norm_hbm.at[tid], normv)

        for p in range(NCCH // NC):
            cch = core * (NCCH // NC) + p

            def _zrow(i, _):
                for j in range(CCH // L):
                    wb[i, pl.ds(j * L, L)] = jnp.zeros((L,), jnp.float32)
                return 0
            lax.fori_loop(0, WB, _zrow, 0)
            def _zero(i, _):
                pltpu.sync_copy(wb, acc.at[pl.ds(tid * RPT + i * WB, WB)])
                return 0
            lax.fori_loop(0, RPT // WB, _zero, 0)

            pltpu.sync_copy(src_hbm.at[tid], srcc)

            def _mkidx(i, _):
                for j in range(C // L):
                    srcc[i, pl.ds(j * L, L)] = (
                        srcc[i, pl.ds(j * L, L)] + cch * N)
                return 0
            lax.fori_loop(0, NCH, _mkidx, 0)

            plsc.subcore_barrier()

            NG = NCH // KB

            for b in range(KB):
                pltpu.async_copy(h4_hbm.at[srcc.at[b]], rows.at[b], sem)

            def _group(g, _):
                half = lax.rem(g, 2) * KB
                nhalf = lax.rem(g + 1, 2) * KB

                @pl.when(g > 0)
                def _drain_sc():
                    for b in range(KB):
                        pltpu.make_async_copy(
                            rows.at[nhalf + b], acc.at[pl.ds(0, C)],
                            sem2).wait()

                @pl.when(g < NG - 1)
                def _fire():
                    for b in range(KB):
                        ch = (g + 1) * KB + b
                        pltpu.async_copy(
                            h4_hbm.at[srcc.at[ch]], rows.at[nhalf + b], sem)

                for b in range(KB):
                    pltpu.make_async_copy(
                        h4_hbm.at[pl.ds(0, C)], rows.at[half + b], sem).wait()

                for b in range(KB):
                    ch = g * KB + b

                    def _scale(rg2, _, _b=b, _ch=ch, _half=half):
                      for rg in (rg2 * 2, rg2 * 2 + 1):
                        nv = normv[pl.ds(_ch * C + rg * L, L)]
                        for ri in range(L):
                            nrm = lax.gather(
                                nv, jnp.full((L, 1), ri, jnp.int32),
                                lax.GatherDimensionNumbers(
                                    offset_dims=(), collapsed_slice_dims=(0,),
                                    start_index_map=(0,)),
                                (1,),
                                mode=lax.GatherScatterMode.PROMISE_IN_BOUNDS)
                            r = rg * L + ri
                            for j in range(CCH // L):
                                rows[_half + _b, r, pl.ds(j * L, L)] = (
                                    rows[_half + _b, r, pl.ds(j * L, L)] * nrm)
                      return 0
                    lax.fori_loop(0, C // L // 2, _scale, 0)
                for b in range(KB):
                    ch = g * KB + b
                    pltpu.async_copy(
                        rows.at[half + b], acc.at[gidx.at[ch]], sem2,
                        add=True)
                return 0
            lax.fori_loop(0, NG, _group, 0)

            for b in range(KB):
                pltpu.make_async_copy(
                    rows.at[b], acc.at[pl.ds(0, C)], sem2).wait()

            plsc.subcore_barrier()

            def _wback(i, _):
                pltpu.sync_copy(acc.at[pl.ds(tid * RPT + i * WB, WB)], wb)
                pltpu.sync_copy(
                    wb, out_hbm.at[pl.ds(tid * RPT + i * WB, WB),
                                   pl.ds(cch * CCH, CCH)])
                return 0
            lax.fori_loop(0, RPT // WB, _wback, 0)

            plsc.subcore_barrier()

    return k(h4, src_r, gidx_r, norm_r)


def _bn_relu(y, g, b):
    m = jnp.mean(y, axis=0, keepdims=True)
    v = jnp.mean((y - m) ** 2, axis=0, keepdims=True)
    return jnp.maximum((y - m) * lax.rsqrt(v + EPS) * g + b, 0.0)


BM = 2000


def _mm_bias(x, w, b):
    M, K = x.shape
    F = w.shape[1]

    def body(x_ref, w_ref, b_ref, o_ref):
        o_ref[...] = jnp.dot(
            x_ref[...], w_ref[...], preferred_element_type=jnp.float32,
            precision=lax.Precision.HIGHEST) + b_ref[...]

    return pl.pallas_call(
        body,
        grid=(M // BM,),
        in_specs=[
            pl.BlockSpec((BM, K), lambda i: (i, 0)),
            pl.BlockSpec((K, F), lambda i: (0, 0)),
            pl.BlockSpec((1, F), lambda i: (0, 0)),
        ],
        out_specs=pl.BlockSpec((BM, F), lambda i: (i, 0)),
        out_shape=jax.ShapeDtypeStruct((M, F), jnp.float32),
    )(x, w, b)


def _gw(G, W):
    def body(g0_ref, g1_ref, g2_ref, w_ref, o_ref):
        acc = jnp.dot(g0_ref[...], w_ref[0],
                      preferred_element_type=jnp.float32,
                      precision=lax.Precision.HIGHEST)
        acc = acc + jnp.dot(g1_ref[...], w_ref[1],
                            preferred_element_type=jnp.float32,
                            precision=lax.Precision.HIGHEST)
        acc = acc + jnp.dot(g2_ref[...], w_ref[2],
                            preferred_element_type=jnp.float32,
                            precision=lax.Precision.HIGHEST)
        o_ref[...] = acc

    nb = N // BM
    gspec = lambda r: pl.BlockSpec((BM, D), lambda i, _r=r: (_r * nb + i, 0))
    return pl.pallas_call(
        body,
        grid=(nb,),
        in_specs=[gspec(0), gspec(1), gspec(2),
                  pl.BlockSpec((3, D, D), lambda i: (0, 0, 0))],
        out_specs=pl.BlockSpec((BM, D), lambda i: (i, 0)),
        out_shape=jax.ShapeDtypeStruct((N, D), jnp.float32),
    )(G, G, G, W)


BC = 128


def _bnrelu(y, g, b):
    M, F = y.shape

    def body(y_ref, g_ref, bb_ref, o_ref):
        o_ref[...] = _bn_relu(y_ref[...], g_ref[...], bb_ref[...])

    return pl.pallas_call(
        body,
        grid=(F // BC,),
        in_specs=[pl.BlockSpec((M, BC), lambda i: (0, i)),
                  pl.BlockSpec((1, BC), lambda i: (0, i)),
                  pl.BlockSpec((1, BC), lambda i: (0, i))],
        out_specs=pl.BlockSpec((M, BC), lambda i: (0, i)),
        out_shape=jax.ShapeDtypeStruct((M, F), jnp.float32),
    )(y, g, b)


def _post_bn(h, s, g, b):
    def body(h_ref, s_ref, g_ref, bb_ref, o_ref):
        swh = s_ref[...]
        y = h_ref[...] + swh
        m = jnp.mean(y, axis=0, keepdims=True)
        v = jnp.mean((y - m) ** 2, axis=0, keepdims=True)
        yn = (y - m) * lax.rsqrt(v + EPS) * g_ref[...] + bb_ref[...]
        o_ref[...] = jnp.maximum(yn + swh, 0.0)

    return pl.pallas_call(
        body,
        grid=(D // BC,),
        in_specs=[pl.BlockSpec((N, BC), lambda i: (0, i)),
                  pl.BlockSpec((N, BC), lambda i: (0, i)),
                  pl.BlockSpec((1, BC), lambda i: (0, i)),
                  pl.BlockSpec((1, BC), lambda i: (0, i))],
        out_specs=pl.BlockSpec((N, BC), lambda i: (0, i)),
        out_shape=jax.ShapeDtypeStruct((N, D), jnp.float32),
    )(h, s, g, b)


def kernel(x, edge_index, rel_type, norm, W_emb, b_emb, bn0_g, bn0_b,
           W_rgcn1, bn1_g, bn1_b, W_rgcn2, bn2_g, bn2_b, Wa1, ba1, bna_g,
           bna_b, Wa2, ba2, Wb1, bb1, bnb_g, bnb_b, Wb2, bb2):
    src = edge_index[0]
    dst = edge_index[1]

    pad = EP - E
    gidx = rel_type * N + dst
    srcp = jnp.concatenate([src, jnp.zeros((pad,), jnp.int32)]).reshape(NS, NCH, C)
    gidxp = jnp.concatenate([gidx, jnp.zeros((pad,), jnp.int32)]).reshape(NS, NCH, C)
    normp = jnp.concatenate([norm, jnp.zeros((pad,), jnp.float32)]).reshape(NS, NCH * C)

    row = lambda a: a.reshape(1, -1)

    y0 = _mm_bias(x, W_emb, row(b_emb))
    h = _bnrelu(y0, row(bn0_g), row(bn0_b))

    for W, g, b in ((W_rgcn1, bn1_g, bn1_b), (W_rgcn2, bn2_g, bn2_b)):
        h4 = h.reshape(N, NCCH, CCH).transpose(1, 0, 2).reshape(NCCH * N, CCH)
        G = _sc_edge_pass(h4, srcp, gidxp, normp)
        swh = _gw(G, W)
        h = _post_bn(h, swh, row(g), row(b))

    W1c = jnp.concatenate([Wa1, Wb1], axis=1)
    b1c = jnp.concatenate([ba1, bb1])
    g1c = jnp.concatenate([bna_g, bnb_g])
    bb1c = jnp.concatenate([bna_b, bnb_b])
    z2 = jnp.zeros((D, 128), jnp.float32)
    Wa2p = z2.at[:, :Wa2.shape[1]].set(Wa2)
    Wb2p = z2.at[:, :Wb2.shape[1]].set(Wb2)
    W2bd = jnp.concatenate([
        jnp.concatenate([Wa2p, jnp.zeros((D, 128), jnp.float32)], axis=1),
        jnp.concatenate([jnp.zeros((D, 128), jnp.float32), Wb2p], axis=1),
    ], axis=0)

    y1 = _mm_bias(h, W1c, row(b1c))
    y2 = _bnrelu(y1, row(g1c), row(bb1c))
    out = _mm_bias(y2, W2bd, jnp.zeros((1, 256), jnp.float32))

    xa = out[:, :2] + ba2
    xb = out[:, 128:144] + bb2
    return (xa, xb)


# speedup vs baseline: 1.3202x; 1.3202x over previous
import functools

import jax
import jax.numpy as jnp
from jax import lax
from jax.experimental import pallas as pl
from jax.experimental.pallas import tpu as pltpu
from jax.experimental.pallas import tpu_sc as plsc

N = 10000
E = 160000
D = 256
EPS = 1e-5

NC = 2
NS = 16
L = 16

CCH = 32
NCCH = D // CCH
C = 128
KB = 4
NCH = 80
EP = NS * NCH * C
TROWS = 30720
RPT = TROWS // NS
WB = 128


def _sc_edge_pass(h4, src_r, gidx_r, norm_r):
    mesh = plsc.VectorSubcoreMesh(
        core_axis_name="c", subcore_axis_name="s",
        num_cores=NC, num_subcores=NS)

    @functools.partial(
        pl.kernel,
        out_type=jax.ShapeDtypeStruct((TROWS, D), jnp.float32),
        mesh=mesh,
        scratch_types=[
            pltpu.VMEM((NCH, C), jnp.int32),
            pltpu.VMEM((NCH, C), jnp.int32),
            pltpu.VMEM((NCH * C,), jnp.float32),
            pltpu.VMEM((2 * KB, C, CCH), jnp.float32),
            pltpu.VMEM((WB, CCH), jnp.float32),
            pltpu.VMEM_SHARED((TROWS, CCH), jnp.float32),
            pltpu.SemaphoreType.DMA,
            pltpu.SemaphoreType.DMA,
        ],
        compiler_params=pltpu.CompilerParams(use_tc_tiling_on_sc=False),
    )
    def k(h4_hbm, src_hbm, gidx_hbm, norm_hbm, out_hbm,
          srcc, gidx, normv, rows, wb, acc, sem, sem2):
        core = lax.axis_index("c")
        tid = lax.axis_index("s")

        pltpu.sync_copy(gidx_hbm.at[tid], gidx)
        pltpu.sync_copy(norm_hbm.at[tid], normv)

        for p in range(NCCH // NC):
            cch = core * (NCCH // NC) + p

            def _zrow(i, _):
                for j in range(CCH // L):
                    wb[i, pl.ds(j * L, L)] = jnp.zeros((L,), jnp.float32)
                return 0
            lax.fori_loop(0, WB, _zrow, 0)
            def _zero(i, _):
                pltpu.sync_copy(wb, acc.at[pl.ds(tid * RPT + i * WB, WB)])
                return 0
            lax.fori_loop(0, RPT // WB, _zero, 0)

            pltpu.sync_copy(src_hbm.at[tid], srcc)

            def _mkidx(i, _):
                for j in range(C // L):
                    srcc[i, pl.ds(j * L, L)] = (
                        srcc[i, pl.ds(j * L, L)] + cch * N)
                return 0
            lax.fori_loop(0, NCH, _mkidx, 0)

            plsc.subcore_barrier()

            NG = NCH // KB

            for b in range(KB):
                pltpu.async_copy(h4_hbm.at[srcc.at[b]], rows.at[b], sem)

            def _group(g, _):
                half = lax.rem(g, 2) * KB
                nhalf = lax.rem(g + 1, 2) * KB

                @pl.when(g > 0)
                def _drain_sc():
                    for b in range(KB):
                        pltpu.make_async_copy(
                            rows.at[nhalf + b], acc.at[pl.ds(0, C)],
                            sem2).wait()

                @pl.when(g < NG - 1)
                def _fire():
                    for b in range(KB):
                        ch = (g + 1) * KB + b
                        pltpu.async_copy(
                            h4_hbm.at[srcc.at[ch]], rows.at[nhalf + b], sem)

                for b in range(KB):
                    pltpu.make_async_copy(
                        h4_hbm.at[pl.ds(0, C)], rows.at[half + b], sem).wait()

                for b in range(KB):
                    ch = g * KB + b

                    def _scale(rg, _, _b=b, _ch=ch, _half=half):
                        nv = normv[pl.ds(_ch * C + rg * L, L)]
                        for ri in range(L):
                            nrm = lax.gather(
                                nv, jnp.full((L, 1), ri, jnp.int32),
                                lax.GatherDimensionNumbers(
                                    offset_dims=(), collapsed_slice_dims=(0,),
                                    start_index_map=(0,)),
                                (1,),
                                mode=lax.GatherScatterMode.PROMISE_IN_BOUNDS)
                            r = rg * L + ri
                            for j in range(CCH // L):
                                rows[_half + _b, r, pl.ds(j * L, L)] = (
                                    rows[_half + _b, r, pl.ds(j * L, L)] * nrm)
                        return 0
                    lax.fori_loop(0, C // L, _scale, 0)
                for b in range(KB):
                    ch = g * KB + b
                    pltpu.async_copy(
                        rows.at[half + b], acc.at[gidx.at[ch]], sem2,
                        add=True)
                return 0
            lax.fori_loop(0, NG, _group, 0)

            for b in range(KB):
                pltpu.make_async_copy(
                    rows.at[b], acc.at[pl.ds(0, C)], sem2).wait()

            plsc.subcore_barrier()

            def _wback(i, _):
                pltpu.sync_copy(acc.at[pl.ds(tid * RPT + i * WB, WB)], wb)
                pltpu.sync_copy(
                    wb, out_hbm.at[pl.ds(tid * RPT + i * WB, WB),
                                   pl.ds(cch * CCH, CCH)])
                return 0
            lax.fori_loop(0, RPT // WB, _wback, 0)

            plsc.subcore_barrier()

    return k(h4, src_r, gidx_r, norm_r)


def _bn_relu(y, g, b):
    m = jnp.mean(y, axis=0, keepdims=True)
    v = jnp.mean((y - m) ** 2, axis=0, keepdims=True)
    return jnp.maximum((y - m) * lax.rsqrt(v + EPS) * g + b, 0.0)


BM = 2000


def _mm_bias(x, w, b):
    M, K = x.shape
    F = w.shape[1]

    def body(x_ref, w_ref, b_ref, o_ref):
        o_ref[...] = jnp.dot(
            x_ref[...], w_ref[...], preferred_element_type=jnp.float32,
            precision=lax.Precision.HIGHEST) + b_ref[...]

    return pl.pallas_call(
        body,
        grid=(M // BM,),
        in_specs=[
            pl.BlockSpec((BM, K), lambda i: (i, 0)),
            pl.BlockSpec((K, F), lambda i: (0, 0)),
            pl.BlockSpec((1, F), lambda i: (0, 0)),
        ],
        out_specs=pl.BlockSpec((BM, F), lambda i: (i, 0)),
        out_shape=jax.ShapeDtypeStruct((M, F), jnp.float32),
    )(x, w, b)


def _gw(G, W):
    def body(g0_ref, g1_ref, g2_ref, w_ref, o_ref):
        acc = jnp.dot(g0_ref[...], w_ref[0],
                      preferred_element_type=jnp.float32,
                      precision=lax.Precision.HIGHEST)
        acc = acc + jnp.dot(g1_ref[...], w_ref[1],
                            preferred_element_type=jnp.float32,
                            precision=lax.Precision.HIGHEST)
        acc = acc + jnp.dot(g2_ref[...], w_ref[2],
                            preferred_element_type=jnp.float32,
                            precision=lax.Precision.HIGHEST)
        o_ref[...] = acc

    nb = N // BM
    gspec = lambda r: pl.BlockSpec((BM, D), lambda i, _r=r: (_r * nb + i, 0))
    return pl.pallas_call(
        body,
        grid=(nb,),
        in_specs=[gspec(0), gspec(1), gspec(2),
                  pl.BlockSpec((3, D, D), lambda i: (0, 0, 0))],
        out_specs=pl.BlockSpec((BM, D), lambda i: (i, 0)),
        out_shape=jax.ShapeDtypeStruct((N, D), jnp.float32),
    )(G, G, G, W)


BC = 128


def _bnrelu(y, g, b):
    M, F = y.shape

    def body(y_ref, g_ref, bb_ref, o_ref):
        o_ref[...] = _bn_relu(y_ref[...], g_ref[...], bb_ref[...])

    return pl.pallas_call(
        body,
        grid=(F // BC,),
        in_specs=[pl.BlockSpec((M, BC), lambda i: (0, i)),
                  pl.BlockSpec((1, BC), lambda i: (0, i)),
                  pl.BlockSpec((1, BC), lambda i: (0, i))],
        out_specs=pl.BlockSpec((M, BC), lambda i: (0, i)),
        out_shape=jax.ShapeDtypeStruct((M, F), jnp.float32),
    )(y, g, b)


def _post_bn(h, s, g, b):
    def body(h_ref, s_ref, g_ref, bb_ref, o_ref):
        swh = s_ref[...]
        y = h_ref[...] + swh
        m = jnp.mean(y, axis=0, keepdims=True)
        v = jnp.mean((y - m) ** 2, axis=0, keepdims=True)
        yn = (y - m) * lax.rsqrt(v + EPS) * g_ref[...] + bb_ref[...]
        o_ref[...] = jnp.maximum(yn + swh, 0.0)

    return pl.pallas_call(
        body,
        grid=(D // BC,),
        in_specs=[pl.BlockSpec((N, BC), lambda i: (0, i)),
                  pl.BlockSpec((N, BC), lambda i: (0, i)),
                  pl.BlockSpec((1, BC), lambda i: (0, i)),
                  pl.BlockSpec((1, BC), lambda i: (0, i))],
        out_specs=pl.BlockSpec((N, BC), lambda i: (0, i)),
        out_shape=jax.ShapeDtypeStruct((N, D), jnp.float32),
    )(h, s, g, b)


def kernel(x, edge_index, rel_type, norm, W_emb, b_emb, bn0_g, bn0_b,
           W_rgcn1, bn1_g, bn1_b, W_rgcn2, bn2_g, bn2_b, Wa1, ba1, bna_g,
           bna_b, Wa2, ba2, Wb1, bb1, bnb_g, bnb_b, Wb2, bb2):
    src = edge_index[0]
    dst = edge_index[1]

    pad = EP - E
    gidx = rel_type * N + dst
    srcp = jnp.concatenate([src, jnp.zeros((pad,), jnp.int32)]).reshape(NS, NCH, C)
    gidxp = jnp.concatenate([gidx, jnp.zeros((pad,), jnp.int32)]).reshape(NS, NCH, C)
    normp = jnp.concatenate([norm, jnp.zeros((pad,), jnp.float32)]).reshape(NS, NCH * C)

    row = lambda a: a.reshape(1, -1)

    y0 = _mm_bias(x, W_emb, row(b_emb))
    h = _bnrelu(y0, row(bn0_g), row(bn0_b))

    for W, g, b in ((W_rgcn1, bn1_g, bn1_b), (W_rgcn2, bn2_g, bn2_b)):
        h4 = h.reshape(N, NCCH, CCH).transpose(1, 0, 2).reshape(NCCH * N, CCH)
        G = _sc_edge_pass(h4, srcp, gidxp, normp)
        swh = _gw(G, W)
        h = _post_bn(h, swh, row(g), row(b))

    W1c = jnp.concatenate([Wa1, Wb1], axis=1)
    b1c = jnp.concatenate([ba1, bb1])
    g1c = jnp.concatenate([bna_g, bnb_g])
    bb1c = jnp.concatenate([bna_b, bnb_b])
    z2 = jnp.zeros((D, 128), jnp.float32)
    Wa2p = z2.at[:, :Wa2.shape[1]].set(Wa2)
    Wb2p = z2.at[:, :Wb2.shape[1]].set(Wb2)
    W2bd = jnp.concatenate([
        jnp.concatenate([Wa2p, jnp.zeros((D, 128), jnp.float32)], axis=1),
        jnp.concatenate([jnp.zeros((D, 128), jnp.float32), Wb2p], axis=1),
    ], axis=0)

    y1 = _mm_bias(h, W1c, row(b1c))
    y2 = _bnrelu(y1, row(g1c), row(bb1c))
    out = _mm_bias(y2, W2bd, jnp.zeros((1, 256), jnp.float32))

    xa = out[:, :2] + ba2
    xb = out[:, 128:144] + bb2
    return (xa, xb)
